# Initial kernel scaffold; baseline (speedup 1.0000x reference)
import jax
import jax.numpy as jnp
from jax.experimental import pallas as pl


def kernel(pos1, pos2, feats1, feats2, factor):
    raise NotImplementedError("write your pallas kernel here")


# speedup vs baseline: 924.9276x; 924.9276x over previous
import functools

import jax
import jax.numpy as jnp
from jax import lax
from jax.experimental import pallas as pl
from jax.experimental.pallas import tpu as pltpu
from jax.experimental.pallas import tpu_sc as plsc

NSAMPLE = 16
RADIUS = 0.2
_BIG = 1 << 20


def _ball_query_body(tq, q_ref, c1_ref, c2_ref, f_ref, out_ref):
    b = pl.program_id(0)
    q = q_ref[0]
    qsq = jnp.sum(q * q, axis=1, keepdims=True)
    r2 = jnp.float32(RADIUS ** 2)
    iota = lax.broadcasted_iota(jnp.int32, (tq, NSAMPLE), 1)
    factor = f_ref[0]

    def sorted16(c_ref):
        cand = c_ref[0]
        csq = jnp.sum(cand * cand, axis=0, keepdims=True)
        d = -2.0 * lax.dot_general(q, cand, (((1,), (0,)), ((), ())),
                                   preferred_element_type=jnp.float32)
        d = d + qsq
        d = d + csq
        keep = jnp.logical_not(d > r2)
        v = jnp.where(keep, iota, _BIG)
        mins = []
        for i in range(NSAMPLE):
            m = jnp.min(v, axis=1, keepdims=True)
            mins.append(m)
            if i + 1 < NSAMPLE:
                v = jnp.where(v == m, _BIG, v)
        pad = jnp.full((tq, 1), NSAMPLE, jnp.int32)
        first = jnp.where(mins[0] < _BIG, mins[0], pad)
        return jnp.concatenate(
            [jnp.where(m < _BIG, m, first) for m in mins], axis=1)

    self16 = sorted16(c1_ref)
    intra16 = sorted16(c2_ref)

    k_row = lax.broadcasted_iota(jnp.int32, (NSAMPLE, NSAMPLE), 0)
    k_col = lax.broadcasted_iota(jnp.int32, (NSAMPLE, NSAMPLE), 1)
    shift = jnp.clip(k_col - factor, 0, NSAMPLE - 1)
    perm = (k_row == shift).astype(jnp.float32)
    intra_sh = lax.dot_general(intra16.astype(jnp.float32), perm,
                               (((1,), (0,)), ((), ())),
                               preferred_element_type=jnp.float32)
    intra_sh = intra_sh.astype(jnp.int32)

    sel = iota < factor
    base = (b * 2 * (NSAMPLE + 1)).astype(jnp.int32)
    out_ref[0] = jnp.where(sel, self16, intra_sh + (NSAMPLE + 1)) + base


def _ball_query(pos1_t, c1, c2, factor):
    b, s, _ = pos1_t.shape
    tq = 512
    body = functools.partial(_ball_query_body, tq)
    factor_arr = jnp.asarray(factor, jnp.int32).reshape(1)
    return pl.pallas_call(
        body,
        grid=(b, s // tq),
        in_specs=[
            pl.BlockSpec((1, tq, 3), lambda i, j: (i, j, 0)),
            pl.BlockSpec((1, 3, NSAMPLE), lambda i, j: (i, 0, 0)),
            pl.BlockSpec((1, 3, NSAMPLE), lambda i, j: (i, 0, 0)),
            pl.BlockSpec(memory_space=pltpu.MemorySpace.SMEM),
        ],
        out_specs=pl.BlockSpec((1, tq, NSAMPLE), lambda i, j: (i, j, 0)),
        out_shape=jax.ShapeDtypeStruct((b, s, NSAMPLE), jnp.int32),
    )(pos1_t, c1, c2, factor_arr)


def _make_sc_gather(rows_total, fdim, pdim):
    info = plsc.get_sparse_core_info()
    nc, ns = info.num_cores, info.num_subcores
    nw = nc * ns
    rows_per_w = rows_total // nw
    chunk = 1024
    n_chunks = rows_per_w // chunk
    g_per_chunk = chunk // 128
    mesh = plsc.VectorSubcoreMesh(core_axis_name="c", subcore_axis_name="s")

    @functools.partial(
        pl.kernel, mesh=mesh,
        compiler_params=pltpu.CompilerParams(use_tc_tiling_on_sc=False),
        out_type=[
            jax.ShapeDtypeStruct((rows_total, fdim), jnp.float32),
            jax.ShapeDtypeStruct((rows_total, pdim), jnp.float32),
        ],
        scratch_types=[
            pltpu.VMEM((g_per_chunk, 128), jnp.int32),
            pltpu.VMEM((chunk, fdim), jnp.float32),
            pltpu.VMEM((chunk, pdim), jnp.float32),
            pltpu.SemaphoreType.DMA,
            pltpu.SemaphoreType.DMA,
        ],
    )
    def sc_gather(idx_hbm, fsrc_hbm, psrc_hbm, fout_hbm, pout_hbm,
                  idx_v, fbuf, pbuf, semf, semp):
        wid = lax.axis_index("s") * nc + lax.axis_index("c")

        def chunk_body(c, carry):
            r0 = wid * (rows_per_w // 128) + c * g_per_chunk
            pltpu.sync_copy(idx_hbm.at[pl.ds(r0, g_per_chunk)], idx_v)
            cps = []
            for g in range(g_per_chunk):
                cps.append(pltpu.async_copy(
                    fsrc_hbm.at[idx_v.at[g]],
                    fbuf.at[pl.ds(g * 128, 128)], semf))
                cps.append(pltpu.async_copy(
                    psrc_hbm.at[idx_v.at[g]],
                    pbuf.at[pl.ds(g * 128, 128)], semp))
            for cp in cps:
                cp.wait()
            out0 = wid * rows_per_w + c * chunk
            pltpu.sync_copy(fbuf, fout_hbm.at[pl.ds(out0, chunk)])
            pltpu.sync_copy(pbuf, pout_hbm.at[pl.ds(out0, chunk)])
            return carry

        lax.fori_loop(0, n_chunks, chunk_body, 0)

    return sc_gather


def kernel(pos1, pos2, feats1, feats2, factor):
    b, c, n = feats1.shape
    s = pos1.shape[2]
    t = NSAMPLE + 1
    pos1_t = jnp.transpose(pos1, (0, 2, 1))

    cidx = _ball_query(pos1_t, pos1[:, :, :NSAMPLE], pos2[:, :, :NSAMPLE],
                       factor)

    fsrc = jnp.concatenate([feats1[:, :, :t], feats2[:, :, :t]], axis=2)
    fsrc = jnp.transpose(fsrc, (0, 2, 1)).reshape(b * 2 * t, c)
    pdim = 16
    psrc = jnp.concatenate([pos1[:, :, :t], pos2[:, :, :t]], axis=2)
    psrc = jnp.transpose(psrc, (0, 2, 1))
    psrc = jnp.concatenate(
        [psrc, jnp.zeros((b, 2 * t, pdim - 3), jnp.float32)], axis=2)
    psrc = psrc.reshape(b * 2 * t, pdim)

    rows_total = b * s * NSAMPLE
    idx2d = cidx.reshape(rows_total // 128, 128)

    frows, prows = _make_sc_gather(rows_total, c, pdim)(idx2d, fsrc, psrc)

    new_feats1 = frows.reshape(b, s, NSAMPLE, c).transpose(0, 3, 1, 2)
    new_pos1 = prows.reshape(b, s, NSAMPLE, pdim)[..., :3].transpose(0, 3, 1, 2)
    return new_pos1, new_feats1


# speedup vs baseline: 1344.4072x; 1.4535x over previous
import functools

import jax
import jax.numpy as jnp
from jax import lax
from jax.experimental import pallas as pl
from jax.experimental.pallas import tpu as pltpu
from jax.experimental.pallas import tpu_sc as plsc

NSAMPLE = 16
RADIUS = 0.2
_BIG = 1 << 20


def _ball_query_body(tq, q_ref, c1_ref, c2_ref, f_ref, out_ref):
    b = pl.program_id(0)
    q = q_ref[0]
    qsq = jnp.sum(q * q, axis=1, keepdims=True)
    r2 = jnp.float32(RADIUS ** 2)
    iota = lax.broadcasted_iota(jnp.int32, (tq, NSAMPLE), 1)
    factor = f_ref[0]

    def sorted16(c_ref):
        cand = c_ref[0]
        csq = jnp.sum(cand * cand, axis=0, keepdims=True)
        d = -2.0 * lax.dot_general(q, cand, (((1,), (0,)), ((), ())),
                                   preferred_element_type=jnp.float32)
        d = d + qsq
        d = d + csq
        keep = jnp.logical_not(d > r2)
        v = jnp.where(keep, iota, _BIG)
        mins = []
        for i in range(NSAMPLE):
            m = jnp.min(v, axis=1, keepdims=True)
            mins.append(m)
            if i + 1 < NSAMPLE:
                v = jnp.where(v == m, _BIG, v)
        pad = jnp.full((tq, 1), NSAMPLE, jnp.int32)
        first = jnp.where(mins[0] < _BIG, mins[0], pad)
        return jnp.concatenate(
            [jnp.where(m < _BIG, m, first) for m in mins], axis=1)

    self16 = sorted16(c1_ref)
    intra16 = sorted16(c2_ref)

    k_row = lax.broadcasted_iota(jnp.int32, (NSAMPLE, NSAMPLE), 0)
    k_col = lax.broadcasted_iota(jnp.int32, (NSAMPLE, NSAMPLE), 1)
    shift = jnp.clip(k_col - factor, 0, NSAMPLE - 1)
    perm = (k_row == shift).astype(jnp.float32)
    intra_sh = lax.dot_general(intra16.astype(jnp.float32), perm,
                               (((1,), (0,)), ((), ())),
                               preferred_element_type=jnp.float32)
    intra_sh = intra_sh.astype(jnp.int32)

    sel = iota < factor
    base = (b * 2 * (NSAMPLE + 1)).astype(jnp.int32)
    out_ref[0] = jnp.where(sel, self16, intra_sh + (NSAMPLE + 1)) + base


def _ball_query(pos1_t, c1, c2, factor):
    b, s, _ = pos1_t.shape
    tq = 512
    body = functools.partial(_ball_query_body, tq)
    factor_arr = jnp.asarray(factor, jnp.int32).reshape(1)
    return pl.pallas_call(
        body,
        grid=(b, s // tq),
        in_specs=[
            pl.BlockSpec((1, tq, 3), lambda i, j: (i, j, 0)),
            pl.BlockSpec((1, 3, NSAMPLE), lambda i, j: (i, 0, 0)),
            pl.BlockSpec((1, 3, NSAMPLE), lambda i, j: (i, 0, 0)),
            pl.BlockSpec(memory_space=pltpu.MemorySpace.SMEM),
        ],
        out_specs=pl.BlockSpec((1, tq, NSAMPLE), lambda i, j: (i, j, 0)),
        out_shape=jax.ShapeDtypeStruct((b, s, NSAMPLE), jnp.int32),
    )(pos1_t, c1, c2, factor_arr)


def _make_sc_gather(batch, fdim, pdim, rows_total, n_tab):
    info = plsc.get_sparse_core_info()
    nc, ns = info.num_cores, info.num_subcores
    nw = nc * ns
    rows_per_w = rows_total // nw
    rows_per_b = rows_total // batch
    chunk = 1024
    n_chunks = rows_per_w // chunk
    groups = chunk // 16
    cdim = fdim + pdim
    mesh = plsc.VectorSubcoreMesh(core_axis_name="c", subcore_axis_name="s")

    @functools.partial(
        pl.kernel, mesh=mesh,
        compiler_params=pltpu.CompilerParams(use_tc_tiling_on_sc=False,
                                             needs_layout_passes=False),
        out_type=[
            jax.ShapeDtypeStruct((batch * fdim * rows_per_b,), jnp.float32),
            jax.ShapeDtypeStruct((batch * pdim * rows_per_b,), jnp.float32),
        ],
        scratch_types=[
            pltpu.VMEM((n_tab * cdim,), jnp.float32),
            pltpu.VMEM((chunk,), jnp.int32),
            pltpu.VMEM((cdim, chunk), jnp.float32),
            pltpu.SemaphoreType.DMA,
        ],
    )
    def sc_gather(idx_hbm, tab_hbm, fout_hbm, pout_hbm,
                  tab_v, idx_v, buf, sem):
        wid = lax.axis_index("s") * nc + lax.axis_index("c")
        pltpu.sync_copy(tab_hbm, tab_v)
        bb = wid // (rows_per_b // rows_per_w)
        r0_w = wid * rows_per_w - bb * rows_per_b

        def chunk_body(j, carry):
            pltpu.sync_copy(
                idx_hbm.at[pl.ds(wid * rows_per_w + j * chunk, chunk)], idx_v)

            def group_body(g, carry2):
                rows16 = idx_v[pl.ds(g * 16, 16)]
                base = rows16 * cdim
                for ch in range(cdim):
                    buf[ch, pl.ds(g * 16, 16)] = plsc.load_gather(
                        tab_v, [base + ch])
                return carry2

            lax.fori_loop(0, groups, group_body, 0)

            r0 = r0_w + j * chunk
            cps = []
            for ch in range(fdim):
                cps.append(pltpu.async_copy(
                    buf.at[ch],
                    fout_hbm.at[pl.ds((bb * fdim + ch) * rows_per_b + r0,
                                      chunk)], sem))
            for ch in range(pdim):
                cps.append(pltpu.async_copy(
                    buf.at[fdim + ch],
                    pout_hbm.at[pl.ds((bb * pdim + ch) * rows_per_b + r0,
                                      chunk)], sem))
            for cp in cps:
                cp.wait()
            return carry

        lax.fori_loop(0, n_chunks, chunk_body, 0)

    return sc_gather


def kernel(pos1, pos2, feats1, feats2, factor):
    b, c, n = feats1.shape
    s = pos1.shape[2]
    t = NSAMPLE + 1
    pos1_t = jnp.transpose(pos1, (0, 2, 1))

    cidx = _ball_query(pos1_t, pos1[:, :, :NSAMPLE], pos2[:, :, :NSAMPLE],
                       factor)

    ftab = jnp.concatenate([feats1[:, :, :t], feats2[:, :, :t]], axis=2)
    ptab = jnp.concatenate([pos1[:, :, :t], pos2[:, :, :t]], axis=2)
    tab = jnp.concatenate([ftab, ptab], axis=1)
    tab = jnp.transpose(tab, (0, 2, 1)).reshape(b * 2 * t * (c + 3))

    rows_total = b * s * NSAMPLE
    idx_flat = cidx.reshape(rows_total)

    fflat, pflat = _make_sc_gather(b, c, 3, rows_total, b * 2 * t)(
        idx_flat, tab)

    new_feats1 = fflat.reshape(b, c, s, NSAMPLE)
    new_pos1 = pflat.reshape(b, 3, s, NSAMPLE)
    return new_pos1, new_feats1


# speedup vs baseline: 1812.1135x; 1.3479x over previous
import functools

import jax
import jax.numpy as jnp
from jax import lax
from jax.experimental import pallas as pl
from jax.experimental.pallas import tpu as pltpu
from jax.experimental.pallas import tpu_sc as plsc

NSAMPLE = 16
RADIUS = 0.2
_BIG = 1 << 20


def _ball_query_body(tq, q_ref, c1_ref, c2_ref, f_ref, out_ref):
    b = pl.program_id(0)
    qc = q_ref[0]
    qsq = jnp.sum(qc * qc, axis=0, keepdims=True)
    r2 = jnp.float32(RADIUS ** 2)
    iota = lax.broadcasted_iota(jnp.int32, (NSAMPLE, tq), 0)
    factor = f_ref[0]

    def sorted16(c_ref):
        cand = c_ref[0]
        csq = jnp.sum(cand * cand, axis=0, keepdims=True)
        d = -2.0 * lax.dot_general(cand, qc, (((0,), (0,)), ((), ())),
                                   preferred_element_type=jnp.float32)
        d = d + qsq
        d = d + csq.reshape(NSAMPLE, 1)
        keep = jnp.logical_not(d > r2)
        v = jnp.where(keep, iota, _BIG)
        mins = []
        for i in range(NSAMPLE):
            m = jnp.min(v, axis=0, keepdims=True)
            mins.append(m)
            if i + 1 < NSAMPLE:
                v = jnp.where(v == m, _BIG, v)
        pad = jnp.full((1, tq), NSAMPLE, jnp.int32)
        first = jnp.where(mins[0] < _BIG, mins[0], pad)
        return jnp.concatenate(
            [jnp.where(m < _BIG, m, first) for m in mins], axis=0)

    self16 = sorted16(c1_ref)
    intra16 = sorted16(c2_ref)

    k_row = lax.broadcasted_iota(jnp.int32, (NSAMPLE, NSAMPLE), 0)
    k_col = lax.broadcasted_iota(jnp.int32, (NSAMPLE, NSAMPLE), 1)
    shift = jnp.clip(k_row - factor, 0, NSAMPLE - 1)
    perm = (k_col == shift).astype(jnp.float32)
    intra_sh = lax.dot_general(perm, intra16.astype(jnp.float32),
                               (((1,), (0,)), ((), ())),
                               preferred_element_type=jnp.float32)
    intra_sh = intra_sh.astype(jnp.int32)

    sel = iota < factor
    base = (b * 2 * (NSAMPLE + 1)).astype(jnp.int32)
    outt = jnp.where(sel, self16, intra_sh + (NSAMPLE + 1)) + base
    out_ref[0] = outt.T


def _ball_query(pos1, c1, c2, factor):
    b, _, s = pos1.shape
    tq = 512
    body = functools.partial(_ball_query_body, tq)
    factor_arr = jnp.asarray(factor, jnp.int32).reshape(1)
    return pl.pallas_call(
        body,
        grid=(b, s // tq),
        in_specs=[
            pl.BlockSpec((1, 3, tq), lambda i, j: (i, 0, j)),
            pl.BlockSpec((1, 3, NSAMPLE), lambda i, j: (i, 0, 0)),
            pl.BlockSpec((1, 3, NSAMPLE), lambda i, j: (i, 0, 0)),
            pl.BlockSpec(memory_space=pltpu.MemorySpace.SMEM),
        ],
        out_specs=pl.BlockSpec((1, tq, NSAMPLE), lambda i, j: (i, j, 0)),
        out_shape=jax.ShapeDtypeStruct((b, s, NSAMPLE), jnp.int32),
    )(pos1, c1, c2, factor_arr)


def _make_sc_gather(batch, fdim, pdim, rows_total, n_tab):
    info = plsc.get_sparse_core_info()
    nc, ns = info.num_cores, info.num_subcores
    nw = nc * ns
    rows_per_w = rows_total // nw
    rows_per_b = rows_total // batch
    chunk = 1024
    n_chunks = rows_per_w // chunk
    groups = chunk // 16
    cdim = fdim + pdim
    mesh = plsc.VectorSubcoreMesh(core_axis_name="c", subcore_axis_name="s")

    @functools.partial(
        pl.kernel, mesh=mesh,
        compiler_params=pltpu.CompilerParams(use_tc_tiling_on_sc=False,
                                             needs_layout_passes=False),
        out_type=[
            jax.ShapeDtypeStruct(
                (batch, fdim, rows_per_b // 16, 16), jnp.float32),
            jax.ShapeDtypeStruct(
                (batch, pdim, rows_per_b // 16, 16), jnp.float32),
        ],
        scratch_types=[
            pltpu.VMEM((n_tab * cdim,), jnp.float32),
            pltpu.VMEM((chunk,), jnp.int32),
            pltpu.VMEM((cdim, chunk // 16, 16), jnp.float32),
            pltpu.SemaphoreType.DMA,
        ],
    )
    def sc_gather(idx_hbm, tab_hbm, fout_hbm, pout_hbm,
                  tab_v, idx_v, buf, sem):
        wid = lax.axis_index("s") * nc + lax.axis_index("c")
        pltpu.sync_copy(tab_hbm, tab_v)
        bb = wid // (rows_per_b // rows_per_w)
        r0_w = wid * rows_per_w - bb * rows_per_b

        def chunk_body(j, carry):
            pltpu.sync_copy(
                idx_hbm.at[pl.ds(wid * rows_per_w + j * chunk, chunk)], idx_v)

            def group_body(g, carry2):
                rows16 = idx_v[pl.ds(g * 16, 16)]
                base = rows16 * cdim
                for ch in range(cdim):
                    buf[ch, g, :] = plsc.load_gather(tab_v, [base + ch])
                return carry2

            lax.fori_loop(0, groups, group_body, 0)

            s0 = (r0_w + j * chunk) // 16
            cps = []
            for ch in range(fdim):
                cps.append(pltpu.async_copy(
                    buf.at[ch],
                    fout_hbm.at[bb, ch, pl.ds(s0, groups), :], sem))
            for ch in range(pdim):
                cps.append(pltpu.async_copy(
                    buf.at[fdim + ch],
                    pout_hbm.at[bb, ch, pl.ds(s0, groups), :], sem))
            for cp in cps:
                cp.wait()
            return carry

        lax.fori_loop(0, n_chunks, chunk_body, 0)

    return sc_gather


def kernel(pos1, pos2, feats1, feats2, factor):
    b, c, n = feats1.shape
    s = pos1.shape[2]
    t = NSAMPLE + 1

    cidx = _ball_query(pos1, pos1[:, :, :NSAMPLE], pos2[:, :, :NSAMPLE],
                       factor)

    ftab = jnp.concatenate([feats1[:, :, :t], feats2[:, :, :t]], axis=2)
    ptab = jnp.concatenate([pos1[:, :, :t], pos2[:, :, :t]], axis=2)
    tab = jnp.concatenate([ftab, ptab], axis=1)
    tab = jnp.transpose(tab, (0, 2, 1)).reshape(b * 2 * t * (c + 3))

    rows_total = b * s * NSAMPLE
    idx_flat = cidx.reshape(rows_total)

    new_feats1, new_pos1 = _make_sc_gather(b, c, 3, rows_total, b * 2 * t)(
        idx_flat, tab)
    return new_pos1, new_feats1


# speedup vs baseline: 2586.3781x; 1.4273x over previous
import functools

import jax
import jax.numpy as jnp
from jax import lax
from jax.experimental import pallas as pl
from jax.experimental.pallas import tpu as pltpu
from jax.experimental.pallas import tpu_sc as plsc

NSAMPLE = 16
RADIUS = 0.2
_BIG = 1 << 20


def _ball_query_body(tq, q_ref, c1_ref, c2_ref, f_ref, out_ref):
    b = pl.program_id(0)
    qc = q_ref[0]
    qsq = jnp.sum(qc * qc, axis=0, keepdims=True)
    r2 = jnp.float32(RADIUS ** 2)
    iota = lax.broadcasted_iota(jnp.int32, (NSAMPLE, tq), 0)
    factor = f_ref[0]

    def sorted16(c_ref):
        cand = c_ref[0]
        csq = jnp.sum(cand * cand, axis=0, keepdims=True)
        d = -2.0 * lax.dot_general(cand, qc, (((0,), (0,)), ((), ())),
                                   preferred_element_type=jnp.float32)
        d = d + qsq
        d = d + csq.reshape(NSAMPLE, 1)
        keep = jnp.logical_not(d > r2)
        v = jnp.where(keep, iota, _BIG)
        mins = []
        for i in range(NSAMPLE):
            m = jnp.min(v, axis=0, keepdims=True)
            mins.append(m)
            if i + 1 < NSAMPLE:
                v = jnp.where(v == m, _BIG, v)
        pad = jnp.full((1, tq), NSAMPLE, jnp.int32)
        first = jnp.where(mins[0] < _BIG, mins[0], pad)
        return jnp.concatenate(
            [jnp.where(m < _BIG, m, first) for m in mins], axis=0)

    self16 = sorted16(c1_ref)
    intra16 = sorted16(c2_ref)

    k_row = lax.broadcasted_iota(jnp.int32, (NSAMPLE, NSAMPLE), 0)
    k_col = lax.broadcasted_iota(jnp.int32, (NSAMPLE, NSAMPLE), 1)
    shift = jnp.clip(k_row - factor, 0, NSAMPLE - 1)
    perm = (k_col == shift).astype(jnp.float32)
    intra_sh = lax.dot_general(perm, intra16.astype(jnp.float32),
                               (((1,), (0,)), ((), ())),
                               preferred_element_type=jnp.float32)
    intra_sh = intra_sh.astype(jnp.int32)

    sel = iota < factor
    base = (b * 2 * (NSAMPLE + 1)).astype(jnp.int32)
    outt = jnp.where(sel, self16, intra_sh + (NSAMPLE + 1)) + base
    out_ref[0] = outt.T


def _ball_query(pos1, c1, c2, factor):
    b, _, s = pos1.shape
    tq = 512
    body = functools.partial(_ball_query_body, tq)
    factor_arr = jnp.asarray(factor, jnp.int32).reshape(1)
    return pl.pallas_call(
        body,
        grid=(b, s // tq),
        in_specs=[
            pl.BlockSpec((1, 3, tq), lambda i, j: (i, 0, j)),
            pl.BlockSpec((1, 3, NSAMPLE), lambda i, j: (i, 0, 0)),
            pl.BlockSpec((1, 3, NSAMPLE), lambda i, j: (i, 0, 0)),
            pl.BlockSpec(memory_space=pltpu.MemorySpace.SMEM),
        ],
        out_specs=pl.BlockSpec((1, tq, NSAMPLE), lambda i, j: (i, j, 0)),
        out_shape=jax.ShapeDtypeStruct((b, s, NSAMPLE), jnp.int32),
    )(pos1, c1, c2, factor_arr)


def _make_sc_gather(batch, fdim, pdim, rows_total, n_tab):
    info = plsc.get_sparse_core_info()
    nc, ns = info.num_cores, info.num_subcores
    nw = nc * ns
    rows_per_w = rows_total // nw
    rows_per_b = rows_total // batch
    s_dim = rows_per_b // NSAMPLE
    chunk = 2048
    n_chunks = rows_per_w // chunk
    groups = chunk // NSAMPLE
    cdim = fdim + pdim
    half = (cdim + 1) // 2
    mesh = plsc.VectorSubcoreMesh(core_axis_name="c", subcore_axis_name="s")

    @functools.partial(
        pl.kernel, mesh=mesh,
        compiler_params=pltpu.CompilerParams(needs_layout_passes=False),
        out_type=[
            jax.ShapeDtypeStruct((batch, fdim, NSAMPLE, s_dim), jnp.float32),
            jax.ShapeDtypeStruct((batch, pdim, NSAMPLE, s_dim), jnp.float32),
        ],
        scratch_types=[
            pltpu.VMEM((n_tab * cdim,), jnp.float32),
            pltpu.VMEM((chunk,), jnp.int32),
            pltpu.VMEM((half, NSAMPLE, 128), jnp.float32),
            pltpu.SemaphoreType.DMA,
        ],
    )
    def sc_gather(idx_hbm, tab_hbm, fout_hbm, pout_hbm,
                  tab_v, idx_v, buf, sem):
        wid = lax.axis_index("s") * nc + lax.axis_index("c")
        pltpu.sync_copy(tab_hbm, tab_v)
        bb = wid // (rows_per_b // rows_per_w)
        r0_w = wid * rows_per_w - bb * rows_per_b
        kvec = lax.iota(jnp.int32, 16)

        def chunk_body(j, carry):
            pltpu.sync_copy(
                idx_hbm.at[pl.ds(wid * rows_per_w + j * chunk, chunk)], idx_v)
            s0 = pl.multiple_of((r0_w + j * chunk) // NSAMPLE, 128)

            for ch0 in (0, half):
                nch = min(half, cdim - ch0)

                def group_body(g, carry2, ch0=ch0, nch=nch):
                    rows16 = idx_v[pl.ds(g * NSAMPLE, NSAMPLE)]
                    base = rows16 * cdim
                    gvec = jnp.full((16,), 0, jnp.int32) + g
                    for ci in range(nch):
                        v = plsc.load_gather(tab_v, [base + (ch0 + ci)])
                        plsc.store_scatter(
                            buf, [jnp.full((16,), ci, jnp.int32), kvec, gvec],
                            v)
                    return carry2

                lax.fori_loop(0, groups, group_body, 0)

                cps = []
                for ci in range(nch):
                    ch = ch0 + ci
                    out = fout_hbm if ch < fdim else pout_hbm
                    och = ch if ch < fdim else ch - fdim
                    for kt in range(NSAMPLE // 8):
                        cps.append(pltpu.async_copy(
                            buf.at[ci, pl.ds(kt * 8, 8), :],
                            out.at[bb, och, pl.ds(kt * 8, 8),
                                   pl.ds(s0, 128)], sem))
                for cp in cps:
                    cp.wait()
            return carry

        lax.fori_loop(0, n_chunks, chunk_body, 0)

    return sc_gather


def kernel(pos1, pos2, feats1, feats2, factor):
    b, c, n = feats1.shape
    s = pos1.shape[2]
    t = NSAMPLE + 1

    cidx = _ball_query(pos1, pos1[:, :, :NSAMPLE], pos2[:, :, :NSAMPLE],
                       factor)

    ftab = jnp.concatenate([feats1[:, :, :t], feats2[:, :, :t]], axis=2)
    ptab = jnp.concatenate([pos1[:, :, :t], pos2[:, :, :t]], axis=2)
    tab = jnp.concatenate([ftab, ptab], axis=1)
    tab = jnp.transpose(tab, (0, 2, 1)).reshape(b * 2 * t * (c + 3))

    rows_total = b * s * NSAMPLE
    idx_flat = cidx.reshape(rows_total)

    fks, pks = _make_sc_gather(b, c, 3, rows_total, b * 2 * t)(idx_flat, tab)
    return pks.transpose(0, 1, 3, 2), fks.transpose(0, 1, 3, 2)
